# baseline (speedup 1.0000x reference)
import functools

import jax
import jax.numpy as jnp
from jax import lax
from jax.experimental import pallas as pl
from jax.experimental.pallas import tpu as pltpu
from jax.experimental.pallas import tpu_sc as plsc

N = 10000
NC = 5000
E = 320000
LD = 128
PD = 3
DE = 144
NP = 5120
NF = 10240
NT = NF + 16
CH = 80
CHE = 100
BLK = 640

_mesh = plsc.VectorSubcoreMesh(
    core_axis_name="c", subcore_axis_name="s", num_cores=2, num_subcores=16)
_sc_params = pltpu.CompilerParams(use_tc_tiling_on_sc=False)


def _zero_fill(ref, rows, d):
  def zb(r, _):
    for dd in range(d // 16):
      ref[r, pl.ds(dd * 16, 16)] = jnp.zeros((16,), jnp.float32)
    return 0
  lax.fori_loop(0, rows, zb, 0)


def _make_relu_pass(d, nacc, n_edges):
  per_tile = n_edges // 32
  n_chunks = per_tile // CHE
  rows_pt = nacc // 16

  def body(a_hbm, b_hbm, gi_hbm, gj_hbm, out_hbm,
           gi_v, gj_v, a0_v, b0_v, a1_v, b1_v, acc_sh,
           sa0, sb0, sa1, sb1):
    c = lax.axis_index("c")
    s = lax.axis_index("s")
    tile = c * 16 + s
    _zero_fill(a0_v, CHE, d)
    for off in range(0, rows_pt, CHE):
      sz = min(CHE, rows_pt - off)
      pltpu.sync_copy(a0_v.at[pl.ds(0, sz)],
                      acc_sh.at[pl.ds(s * rows_pt + off, sz)])
    plsc.subcore_barrier()
    pltpu.sync_copy(gi_hbm.at[tile], gi_v)
    pltpu.sync_copy(gj_hbm.at[tile], gj_v)

    def issue(t, av, bv, sa, sb):
      pltpu.async_copy(a_hbm.at[gi_v.at[t]], av, sa)
      pltpu.async_copy(b_hbm.at[gj_v.at[t]], bv, sb)

    def drain(t, av, bv, sa, sb):
      pltpu.make_async_copy(a_hbm.at[gi_v.at[t]], av, sa).wait()
      pltpu.make_async_copy(b_hbm.at[gj_v.at[t]], bv, sb).wait()

      def cb(r, _):
        for dd in range(d // 16):
          sl = pl.ds(dd * 16, 16)
          av[r, sl] = jnp.maximum(av[r, sl] + bv[r, sl], 0.0)
        return 0
      lax.fori_loop(0, CHE, cb, 0)
      pltpu.sync_copy(av, acc_sh.at[gi_v.at[t]], add=True)

    issue(0, a0_v, b0_v, sa0, sb0)

    def eb(p, _):
      t0 = 2 * p
      issue(t0 + 1, a1_v, b1_v, sa1, sb1)
      drain(t0, a0_v, b0_v, sa0, sb0)

      @pl.when(t0 + 2 < n_chunks)
      def _():
        issue(t0 + 2, a0_v, b0_v, sa0, sb0)
      drain(t0 + 1, a1_v, b1_v, sa1, sb1)
      return 0
    lax.fori_loop(0, n_chunks // 2, eb, 0)
    if n_chunks % 2:
      drain(n_chunks - 1, a0_v, b0_v, sa0, sb0)
    plsc.subcore_barrier()
    pltpu.sync_copy(acc_sh.at[pl.ds(s * rows_pt, rows_pt)],
                    out_hbm.at[c].at[pl.ds(s * rows_pt, rows_pt)])

  return pl.kernel(
      body,
      out_type=jax.ShapeDtypeStruct((2, nacc, d), jnp.float32),
      mesh=_mesh,
      compiler_params=_sc_params,
      scratch_types=[
          pltpu.VMEM((n_chunks, CHE), jnp.int32),
          pltpu.VMEM((n_chunks, CHE), jnp.int32),
          pltpu.VMEM((CHE, d), jnp.float32),
          pltpu.VMEM((CHE, d), jnp.float32),
          pltpu.VMEM((CHE, d), jnp.float32),
          pltpu.VMEM((CHE, d), jnp.float32),
          pltpu.VMEM_SHARED((nacc, d), jnp.float32),
          pltpu.SemaphoreType.DMA,
          pltpu.SemaphoreType.DMA,
          pltpu.SemaphoreType.DMA,
          pltpu.SemaphoreType.DMA,
      ])


def _make_copy_pass(d, nacc, n_edges):
  per_tile = n_edges // 32
  n_chunks = per_tile // CHE
  rows_pt = nacc // 16

  def body(src_hbm, gg_hbm, gs_hbm, out_hbm,
           gg_v, gs_v, r0_v, r1_v, acc_sh, s0, s1):
    c = lax.axis_index("c")
    s = lax.axis_index("s")
    tile = c * 16 + s
    _zero_fill(r0_v, CHE, d)
    for off in range(0, rows_pt, CHE):
      sz = min(CHE, rows_pt - off)
      pltpu.sync_copy(r0_v.at[pl.ds(0, sz)],
                      acc_sh.at[pl.ds(s * rows_pt + off, sz)])
    plsc.subcore_barrier()
    pltpu.sync_copy(gg_hbm.at[tile], gg_v)
    pltpu.sync_copy(gs_hbm.at[tile], gs_v)

    def issue(t, rv, sem):
      pltpu.async_copy(src_hbm.at[gg_v.at[t]], rv, sem)

    def drain(t, rv, sem):
      pltpu.make_async_copy(src_hbm.at[gg_v.at[t]], rv, sem).wait()
      pltpu.sync_copy(rv, acc_sh.at[gs_v.at[t]], add=True)

    issue(0, r0_v, s0)

    def eb(p, _):
      t0 = 2 * p
      issue(t0 + 1, r1_v, s1)
      drain(t0, r0_v, s0)

      @pl.when(t0 + 2 < n_chunks)
      def _():
        issue(t0 + 2, r0_v, s0)
      drain(t0 + 1, r1_v, s1)
      return 0
    lax.fori_loop(0, n_chunks // 2, eb, 0)
    if n_chunks % 2:
      drain(n_chunks - 1, r0_v, s0)
    plsc.subcore_barrier()
    pltpu.sync_copy(acc_sh.at[pl.ds(s * rows_pt, rows_pt)],
                    out_hbm.at[c].at[pl.ds(s * rows_pt, rows_pt)])

  return pl.kernel(
      body,
      out_type=jax.ShapeDtypeStruct((2, nacc, d), jnp.float32),
      mesh=_mesh,
      compiler_params=_sc_params,
      scratch_types=[
          pltpu.VMEM((n_chunks, CHE), jnp.int32),
          pltpu.VMEM((n_chunks, CHE), jnp.int32),
          pltpu.VMEM((CHE, d), jnp.float32),
          pltpu.VMEM((CHE, d), jnp.float32),
          pltpu.VMEM_SHARED((nacc, d), jnp.float32),
          pltpu.SemaphoreType.DMA,
          pltpu.SemaphoreType.DMA,
      ])


def _make_gather(d, n_out, n_tbl):
  per_tile = n_out // 32

  def body(tbl_hbm, ids_hbm, out_hbm, id_v, r_v, sem):
    c = lax.axis_index("c")
    s = lax.axis_index("s")
    base0 = (c * 16 + s) * per_tile
    for q in range(per_tile // CH):
      base = base0 + q * CH
      pltpu.sync_copy(ids_hbm.at[pl.ds(base, CH)], id_v)
      pltpu.async_copy(tbl_hbm.at[id_v], r_v, sem).wait()
      pltpu.sync_copy(r_v, out_hbm.at[pl.ds(base, CH)])

  return pl.kernel(
      body,
      out_type=jax.ShapeDtypeStruct((n_out, d), jnp.float32),
      mesh=_mesh,
      compiler_params=_sc_params,
      scratch_types=[
          pltpu.VMEM((CH,), jnp.int32),
          pltpu.VMEM((CH, d), jnp.float32),
          pltpu.SemaphoreType.DMA,
      ])


def _make_scatter(d, n_src, n_tbl):
  per_tile_src = n_src // 16
  rows_pt = n_tbl // 16

  def body(src_hbm, tgt_hbm, out_hbm, t_v, r_v, z_v, sem):
    c = lax.axis_index("c")
    s = lax.axis_index("s")

    @pl.when(c == 0)
    def _():
      _zero_fill(z_v, rows_pt, d)
      pltpu.sync_copy(z_v, out_hbm.at[pl.ds(s * rows_pt, rows_pt)])
      plsc.subcore_barrier()
      for q in range(per_tile_src // CH):
        base = s * per_tile_src + q * CH
        pltpu.sync_copy(tgt_hbm.at[pl.ds(base, CH)], t_v)
        pltpu.sync_copy(src_hbm.at[pl.ds(base, CH)], r_v)
        pltpu.async_copy(r_v, out_hbm.at[t_v], sem).wait()

  return pl.kernel(
      body,
      out_type=jax.ShapeDtypeStruct((n_tbl, d), jnp.float32),
      mesh=_mesh,
      compiler_params=_sc_params,
      scratch_types=[
          pltpu.VMEM((CH,), jnp.int32),
          pltpu.VMEM((CH, d), jnp.float32),
          pltpu.VMEM((rows_pt, d), jnp.float32),
          pltpu.SemaphoreType.DMA,
      ])


def _dot(a, b):
  return lax.dot_general(a, b, (((1,), (0,)), ((), ())),
                         preferred_element_type=jnp.float32)


def _mm_ab(x, wa, ba, wb, bb):
  def body(x_r, wa_r, ba_r, wb_r, bb_r, a_r, b_r):
    xv = x_r[...]
    a_r[...] = _dot(xv, wa_r[...]) + ba_r[...]
    b_r[...] = _dot(xv, wb_r[...]) + bb_r[...]

  grid = NP // BLK
  return pl.pallas_call(
      body,
      grid=(grid,),
      in_specs=[
          pl.BlockSpec((BLK, DE), lambda r: (r, 0)),
          pl.BlockSpec((DE, DE), lambda r: (0, 0)),
          pl.BlockSpec((1, DE), lambda r: (0, 0)),
          pl.BlockSpec((DE, DE), lambda r: (0, 0)),
          pl.BlockSpec((1, DE), lambda r: (0, 0)),
      ],
      out_specs=[
          pl.BlockSpec((BLK, DE), lambda r: (r, 0)),
          pl.BlockSpec((BLK, DE), lambda r: (r, 0)),
      ],
      out_shape=[jax.ShapeDtypeStruct((NP, DE), jnp.float32)] * 2,
  )(x, wa, ba, wb, bb)


def _node_mlp(sparts, h, res, ew2, eb2, nw1a, nw1b, nb1, nw2, nb2, n):
  with_res = res is not None

  def body(*refs):
    if with_res:
      (s_r, h_r, res_r, ew2_r, eb2_r, w1a_r, w1b_r, nb1_r,
       nw2_r, nb2_r, out_r, deg_r) = refs
    else:
      (s_r, h_r, ew2_r, eb2_r, w1a_r, w1b_r, nb1_r,
       nw2_r, nb2_r, out_r, deg_r) = refs
    sv = s_r[0] + s_r[1]
    sh = sv[:, :LD]
    deg = sv[:, LD:LD + 1]
    agg = _dot(sh, ew2_r[...]) + deg * eb2_r[...]
    hv = h_r[...]
    t = jnp.maximum(_dot(hv, w1a_r[...]) + _dot(agg, w1b_r[...]) + nb1_r[...],
                    0.0)
    o = hv + _dot(t, nw2_r[...]) + nb2_r[...]
    if with_res:
      o = o + res_r[...]
    out_r[...] = o
    deg_r[...] = deg

  grid = n // BLK
  in_specs = [pl.BlockSpec((2, BLK, DE), lambda r: (0, r, 0)),
              pl.BlockSpec((BLK, LD), lambda r: (r, 0))]
  args = [sparts, h]
  if with_res:
    in_specs.append(pl.BlockSpec((BLK, LD), lambda r: (r, 0)))
    args.append(res)
  in_specs += [
      pl.BlockSpec((LD, LD), lambda r: (0, 0)),
      pl.BlockSpec((1, LD), lambda r: (0, 0)),
      pl.BlockSpec((LD, LD), lambda r: (0, 0)),
      pl.BlockSpec((LD, LD), lambda r: (0, 0)),
      pl.BlockSpec((1, LD), lambda r: (0, 0)),
      pl.BlockSpec((LD, LD), lambda r: (0, 0)),
      pl.BlockSpec((1, LD), lambda r: (0, 0)),
  ]
  args += [ew2, eb2, nw1a, nw1b, nb1, nw2, nb2]
  return pl.pallas_call(
      body,
      grid=(grid,),
      in_specs=in_specs,
      out_specs=[pl.BlockSpec((BLK, LD), lambda r: (r, 0)),
                 pl.BlockSpec((BLK, 1), lambda r: (r, 0))],
      out_shape=[jax.ShapeDtypeStruct((n, LD), jnp.float32),
                 jax.ShapeDtypeStruct((n, 1), jnp.float32)],
  )(*args)


def _combine_scale(parts, deg, n, d):
  def body(p_r, d_r, o_r):
    o_r[...] = (p_r[0] + p_r[1]) / jnp.maximum(d_r[...], 1e-12)

  return pl.pallas_call(
      body,
      grid=(n // BLK,),
      in_specs=[pl.BlockSpec((2, BLK, d), lambda r: (0, r, 0)),
                pl.BlockSpec((BLK, 1), lambda r: (r, 0))],
      out_specs=pl.BlockSpec((BLK, d), lambda r: (r, 0)),
      out_shape=jax.ShapeDtypeStruct((n, d), jnp.float32),
  )(parts, deg)


def _row_scale(x, deg, n, d):
  def body(x_r, d_r, o_r):
    o_r[...] = x_r[...] / jnp.maximum(d_r[...], 1e-12)

  return pl.pallas_call(
      body,
      grid=(n // BLK,),
      in_specs=[pl.BlockSpec((BLK, d), lambda r: (r, 0)),
                pl.BlockSpec((BLK, 1), lambda r: (r, 0))],
      out_specs=pl.BlockSpec((BLK, d), lambda r: (r, 0)),
      out_shape=jax.ShapeDtypeStruct((n, d), jnp.float32),
  )(x, deg)


def _edge_weights(eW1, eb1):
  wi, wj, wp = eW1[:LD], eW1[LD:2 * LD], eW1[2 * LD:]
  z16 = jnp.zeros((LD, DE - LD), jnp.float32)
  zp = jnp.zeros((PD, DE - LD), jnp.float32)
  zr = jnp.zeros((DE - LD - PD, DE), jnp.float32)
  wa = jnp.concatenate([
      jnp.concatenate([wi, z16], 1),
      jnp.concatenate([-wp, zp], 1),
      zr,
  ], 0)
  wb = jnp.concatenate([
      jnp.concatenate([wj, z16], 1),
      jnp.concatenate([wp, zp], 1),
      zr,
  ], 0)
  ba = jnp.concatenate([eb1, jnp.ones((1,), jnp.float32),
                        jnp.zeros((DE - LD - 1,), jnp.float32)])[None]
  bb = jnp.zeros((1, DE), jnp.float32)
  return wa, ba, wb, bb


def kernel(h, m_ids, m_gs, pos, down_eW1, down_eb1, down_eW2, down_eb2,
           down_nW1, down_nb1, down_nW2, down_nb2, bottom_eW1, bottom_eb1,
           bottom_eW2, bottom_eb2, bottom_nW1, bottom_nb1, bottom_nW2,
           bottom_nb2, up_eW1, up_eb1, up_eW2, up_eb2, up_nW1, up_nb1,
           up_nW2, up_nb2):
  f32 = jnp.float32
  g0 = m_gs[0]
  g1 = m_gs[1]
  i0, j0 = g0[0], g0[1]
  i1, j1 = g1[0], g1[1]
  ids0 = m_ids[0]

  ids_pad = jnp.concatenate(
      [ids0, jnp.full((NP - NC,), N, jnp.int32)]).astype(jnp.int32)
  wmask = jnp.concatenate([ids0[:-1] != ids0[1:],
                           jnp.ones((1,), bool)])
  tgt = jnp.where(wmask, ids0, NF).astype(jnp.int32)
  tgt = jnp.concatenate([tgt, jnp.full((NP - NC,), NF, jnp.int32)])

  pospad = jnp.concatenate(
      [pos[:NP], jnp.zeros((NP, DE - LD - PD), f32)], 1)
  h_pad = jnp.concatenate([h, jnp.zeros((NF - N, LD), f32)], 0)

  nce = (E // 32) // CHE
  def chunked(ix):
    return ix.astype(jnp.int32).reshape(32, nce, CHE)
  i0c, j0c = chunked(i0), chunked(j0)
  i1c, j1c = chunked(i1), chunked(j1)

  relu_pass = _make_relu_pass(DE, NP, E)
  copy_pass_de = _make_copy_pass(DE, NP, E)
  copy_pass_ld = _make_copy_pass(LD, NP, E)
  gather_de = _make_gather(DE, NP, NF)
  scatter_ld = _make_scatter(LD, NP, NT)

  def pad_parts(sp):
    return jnp.concatenate(
        [sp, jnp.zeros((2, NF - NP, sp.shape[-1]), f32)], 1)

  wa, ba, wb, bb = _edge_weights(down_eW1, down_eb1)
  x0 = jnp.concatenate([h[:NP], pospad], 1)
  a0, b0 = _mm_ab(x0, wa, ba, wb, bb)
  s0 = relu_pass(a0, b0, i0c, j0c)
  h1, deg0 = _node_mlp(pad_parts(s0), h_pad, None, down_eW2, down_eb2[None],
                       down_nW1[:LD], down_nW1[LD:], down_nb1[None],
                       down_nW2, down_nb2[None], NF)

  h1ext = jnp.concatenate([h1[:NP], pospad], 1)
  cparts = copy_pass_de(h1ext, j0c, i0c)
  conv = _combine_scale(cparts, deg0[:NP], NP, DE)
  ptbl = jnp.concatenate([conv, jnp.zeros((NF - NP, DE), f32)], 0)
  pooled = gather_de(ptbl, ids_pad)

  wa1, ba1, wb1, bb1 = _edge_weights(bottom_eW1, bottom_eb1)
  a1, b1 = _mm_ab(pooled, wa1, ba1, wb1, bb1)
  s1 = relu_pass(a1, b1, i1c, j1c)
  hc, _ = _node_mlp(s1, pooled[:, :LD], None, bottom_eW2, bottom_eb2[None],
                    bottom_nW1[:LD], bottom_nW1[LD:], bottom_nb1[None],
                    bottom_nW2, bottom_nb2[None], NP)

  hu0 = scatter_ld(hc, tgt)
  hus = _row_scale(hu0[:NP], deg0[:NP], NP, LD)
  uparts = copy_pass_ld(hus, i0c, j0c)
  ones = jnp.ones((NP, 1), f32)
  u = _combine_scale(uparts, ones, NP, LD)
  u_pad = jnp.concatenate([u, jnp.zeros((NF - NP, LD), f32)], 0)

  wa2, ba2, wb2, bb2 = _edge_weights(up_eW1, up_eb1)
  x2 = jnp.concatenate([u, pospad], 1)
  a2, b2 = _mm_ab(x2, wa2, ba2, wb2, bb2)
  s2 = relu_pass(a2, b2, i0c, j0c)
  out, _ = _node_mlp(pad_parts(s2), u_pad, h1, up_eW2, up_eb2[None],
                     up_nW1[:LD], up_nW1[LD:], up_nb1[None],
                     up_nW2, up_nb2[None], NF)
  return out[:N]


# speedup vs baseline: 1.0101x; 1.0101x over previous
import functools

import jax
import jax.numpy as jnp
from jax import lax
from jax.experimental import pallas as pl
from jax.experimental.pallas import tpu as pltpu
from jax.experimental.pallas import tpu_sc as plsc

N = 10000
NC = 5000
E = 320000
LD = 128
PD = 3
DE = 144
NP = 5120
NF = 10240
NT = NF + 16
CH = 80
CHE = 80
BLK = 640

_mesh = plsc.VectorSubcoreMesh(
    core_axis_name="c", subcore_axis_name="s", num_cores=2, num_subcores=16)
_sc_params = pltpu.CompilerParams(use_tc_tiling_on_sc=False)


def _zero_fill(ref, rows, d):
  def zb(r, _):
    for dd in range(d // 16):
      ref[r, pl.ds(dd * 16, 16)] = jnp.zeros((16,), jnp.float32)
    return 0
  lax.fori_loop(0, rows, zb, 0)


def _make_relu_pass(d, nacc, n_edges):
  per_tile = n_edges // 32
  n_chunks = per_tile // CHE
  rows_pt = nacc // 16

  def body(a_hbm, b_hbm, gi_hbm, gj_hbm, out_hbm,
           gi_v, gj_v, a0_v, b0_v, a1_v, b1_v, acc_sh,
           sa0, sb0, sa1, sb1):
    c = lax.axis_index("c")
    s = lax.axis_index("s")
    tile = c * 16 + s
    _zero_fill(a0_v, CHE, d)
    for off in range(0, rows_pt, CHE):
      sz = min(CHE, rows_pt - off)
      pltpu.sync_copy(a0_v.at[pl.ds(0, sz)],
                      acc_sh.at[pl.ds(s * rows_pt + off, sz)])
    plsc.subcore_barrier()
    pltpu.sync_copy(gi_hbm.at[tile], gi_v)
    pltpu.sync_copy(gj_hbm.at[tile], gj_v)

    def issue(t, av, bv, sa, sb):
      pltpu.async_copy(a_hbm.at[gi_v.at[t]], av, sa)
      pltpu.async_copy(b_hbm.at[gj_v.at[t]], bv, sb)

    def drain(t, av, bv, sa, sb):
      pltpu.make_async_copy(a_hbm.at[gi_v.at[t]], av, sa).wait()
      pltpu.make_async_copy(b_hbm.at[gj_v.at[t]], bv, sb).wait()

      def cb(q, _):
        r = q * 4
        for rr in range(4):
          for dd in range(d // 16):
            sl = pl.ds(dd * 16, 16)
            av[r + rr, sl] = jnp.maximum(av[r + rr, sl] + bv[r + rr, sl], 0.0)
        return 0
      lax.fori_loop(0, CHE // 4, cb, 0)
      pltpu.sync_copy(av, acc_sh.at[gi_v.at[t]], add=True)

    issue(0, a0_v, b0_v, sa0, sb0)

    def eb(p, _):
      t0 = 2 * p
      issue(t0 + 1, a1_v, b1_v, sa1, sb1)
      drain(t0, a0_v, b0_v, sa0, sb0)

      @pl.when(t0 + 2 < n_chunks)
      def _():
        issue(t0 + 2, a0_v, b0_v, sa0, sb0)
      drain(t0 + 1, a1_v, b1_v, sa1, sb1)
      return 0
    lax.fori_loop(0, n_chunks // 2, eb, 0)
    if n_chunks % 2:
      drain(n_chunks - 1, a0_v, b0_v, sa0, sb0)
    plsc.subcore_barrier()
    pltpu.sync_copy(acc_sh.at[pl.ds(s * rows_pt, rows_pt)],
                    out_hbm.at[c].at[pl.ds(s * rows_pt, rows_pt)])

  return pl.kernel(
      body,
      out_type=jax.ShapeDtypeStruct((2, nacc, d), jnp.float32),
      mesh=_mesh,
      compiler_params=_sc_params,
      scratch_types=[
          pltpu.VMEM((n_chunks, CHE), jnp.int32),
          pltpu.VMEM((n_chunks, CHE), jnp.int32),
          pltpu.VMEM((CHE, d), jnp.float32),
          pltpu.VMEM((CHE, d), jnp.float32),
          pltpu.VMEM((CHE, d), jnp.float32),
          pltpu.VMEM((CHE, d), jnp.float32),
          pltpu.VMEM_SHARED((nacc, d), jnp.float32),
          pltpu.SemaphoreType.DMA,
          pltpu.SemaphoreType.DMA,
          pltpu.SemaphoreType.DMA,
          pltpu.SemaphoreType.DMA,
      ])


def _make_copy_pass(d, nacc, n_edges):
  per_tile = n_edges // 32
  n_chunks = per_tile // CHE
  rows_pt = nacc // 16

  def body(src_hbm, gg_hbm, gs_hbm, out_hbm,
           gg_v, gs_v, r0_v, r1_v, acc_sh, s0, s1):
    c = lax.axis_index("c")
    s = lax.axis_index("s")
    tile = c * 16 + s
    _zero_fill(r0_v, CHE, d)
    for off in range(0, rows_pt, CHE):
      sz = min(CHE, rows_pt - off)
      pltpu.sync_copy(r0_v.at[pl.ds(0, sz)],
                      acc_sh.at[pl.ds(s * rows_pt + off, sz)])
    plsc.subcore_barrier()
    pltpu.sync_copy(gg_hbm.at[tile], gg_v)
    pltpu.sync_copy(gs_hbm.at[tile], gs_v)

    def issue(t, rv, sem):
      pltpu.async_copy(src_hbm.at[gg_v.at[t]], rv, sem)

    def drain(t, rv, sem):
      pltpu.make_async_copy(src_hbm.at[gg_v.at[t]], rv, sem).wait()
      pltpu.sync_copy(rv, acc_sh.at[gs_v.at[t]], add=True)

    issue(0, r0_v, s0)

    def eb(p, _):
      t0 = 2 * p
      issue(t0 + 1, r1_v, s1)
      drain(t0, r0_v, s0)

      @pl.when(t0 + 2 < n_chunks)
      def _():
        issue(t0 + 2, r0_v, s0)
      drain(t0 + 1, r1_v, s1)
      return 0
    lax.fori_loop(0, n_chunks // 2, eb, 0)
    if n_chunks % 2:
      drain(n_chunks - 1, r0_v, s0)
    plsc.subcore_barrier()
    pltpu.sync_copy(acc_sh.at[pl.ds(s * rows_pt, rows_pt)],
                    out_hbm.at[c].at[pl.ds(s * rows_pt, rows_pt)])

  return pl.kernel(
      body,
      out_type=jax.ShapeDtypeStruct((2, nacc, d), jnp.float32),
      mesh=_mesh,
      compiler_params=_sc_params,
      scratch_types=[
          pltpu.VMEM((n_chunks, CHE), jnp.int32),
          pltpu.VMEM((n_chunks, CHE), jnp.int32),
          pltpu.VMEM((CHE, d), jnp.float32),
          pltpu.VMEM((CHE, d), jnp.float32),
          pltpu.VMEM_SHARED((nacc, d), jnp.float32),
          pltpu.SemaphoreType.DMA,
          pltpu.SemaphoreType.DMA,
      ])


def _make_gather(d, n_out, n_tbl):
  per_tile = n_out // 32

  def body(tbl_hbm, ids_hbm, out_hbm, id_v, r_v, sem):
    c = lax.axis_index("c")
    s = lax.axis_index("s")
    base0 = (c * 16 + s) * per_tile
    for q in range(per_tile // CH):
      base = base0 + q * CH
      pltpu.sync_copy(ids_hbm.at[pl.ds(base, CH)], id_v)
      pltpu.async_copy(tbl_hbm.at[id_v], r_v, sem).wait()
      pltpu.sync_copy(r_v, out_hbm.at[pl.ds(base, CH)])

  return pl.kernel(
      body,
      out_type=jax.ShapeDtypeStruct((n_out, d), jnp.float32),
      mesh=_mesh,
      compiler_params=_sc_params,
      scratch_types=[
          pltpu.VMEM((CH,), jnp.int32),
          pltpu.VMEM((CH, d), jnp.float32),
          pltpu.SemaphoreType.DMA,
      ])


def _make_scatter(d, n_src, n_tbl):
  per_tile_src = n_src // 16
  rows_pt = n_tbl // 16

  def body(src_hbm, tgt_hbm, out_hbm, t_v, r_v, z_v, sem):
    c = lax.axis_index("c")
    s = lax.axis_index("s")

    @pl.when(c == 0)
    def _():
      _zero_fill(z_v, rows_pt, d)
      pltpu.sync_copy(z_v, out_hbm.at[pl.ds(s * rows_pt, rows_pt)])
      plsc.subcore_barrier()
      for q in range(per_tile_src // CH):
        base = s * per_tile_src + q * CH
        pltpu.sync_copy(tgt_hbm.at[pl.ds(base, CH)], t_v)
        pltpu.sync_copy(src_hbm.at[pl.ds(base, CH)], r_v)
        pltpu.async_copy(r_v, out_hbm.at[t_v], sem).wait()

  return pl.kernel(
      body,
      out_type=jax.ShapeDtypeStruct((n_tbl, d), jnp.float32),
      mesh=_mesh,
      compiler_params=_sc_params,
      scratch_types=[
          pltpu.VMEM((CH,), jnp.int32),
          pltpu.VMEM((CH, d), jnp.float32),
          pltpu.VMEM((rows_pt, d), jnp.float32),
          pltpu.SemaphoreType.DMA,
      ])


def _dot(a, b):
  return lax.dot_general(a, b, (((1,), (0,)), ((), ())),
                         preferred_element_type=jnp.float32)


def _mm_ab(x, wa, ba, wb, bb):
  def body(x_r, wa_r, ba_r, wb_r, bb_r, a_r, b_r):
    xv = x_r[...]
    a_r[...] = _dot(xv, wa_r[...]) + ba_r[...]
    b_r[...] = _dot(xv, wb_r[...]) + bb_r[...]

  grid = NP // BLK
  return pl.pallas_call(
      body,
      grid=(grid,),
      in_specs=[
          pl.BlockSpec((BLK, DE), lambda r: (r, 0)),
          pl.BlockSpec((DE, DE), lambda r: (0, 0)),
          pl.BlockSpec((1, DE), lambda r: (0, 0)),
          pl.BlockSpec((DE, DE), lambda r: (0, 0)),
          pl.BlockSpec((1, DE), lambda r: (0, 0)),
      ],
      out_specs=[
          pl.BlockSpec((BLK, DE), lambda r: (r, 0)),
          pl.BlockSpec((BLK, DE), lambda r: (r, 0)),
      ],
      out_shape=[jax.ShapeDtypeStruct((NP, DE), jnp.float32)] * 2,
  )(x, wa, ba, wb, bb)


def _node_mlp(sparts, h, res, ew2, eb2, nw1a, nw1b, nb1, nw2, nb2, n):
  with_res = res is not None

  def body(*refs):
    if with_res:
      (s_r, h_r, res_r, ew2_r, eb2_r, w1a_r, w1b_r, nb1_r,
       nw2_r, nb2_r, out_r, deg_r) = refs
    else:
      (s_r, h_r, ew2_r, eb2_r, w1a_r, w1b_r, nb1_r,
       nw2_r, nb2_r, out_r, deg_r) = refs
    sv = s_r[0] + s_r[1]
    sh = sv[:, :LD]
    deg = sv[:, LD:LD + 1]
    agg = _dot(sh, ew2_r[...]) + deg * eb2_r[...]
    hv = h_r[...]
    t = jnp.maximum(_dot(hv, w1a_r[...]) + _dot(agg, w1b_r[...]) + nb1_r[...],
                    0.0)
    o = hv + _dot(t, nw2_r[...]) + nb2_r[...]
    if with_res:
      o = o + res_r[...]
    out_r[...] = o
    deg_r[...] = deg

  grid = n // BLK
  in_specs = [pl.BlockSpec((2, BLK, DE), lambda r: (0, r, 0)),
              pl.BlockSpec((BLK, LD), lambda r: (r, 0))]
  args = [sparts, h]
  if with_res:
    in_specs.append(pl.BlockSpec((BLK, LD), lambda r: (r, 0)))
    args.append(res)
  in_specs += [
      pl.BlockSpec((LD, LD), lambda r: (0, 0)),
      pl.BlockSpec((1, LD), lambda r: (0, 0)),
      pl.BlockSpec((LD, LD), lambda r: (0, 0)),
      pl.BlockSpec((LD, LD), lambda r: (0, 0)),
      pl.BlockSpec((1, LD), lambda r: (0, 0)),
      pl.BlockSpec((LD, LD), lambda r: (0, 0)),
      pl.BlockSpec((1, LD), lambda r: (0, 0)),
  ]
  args += [ew2, eb2, nw1a, nw1b, nb1, nw2, nb2]
  return pl.pallas_call(
      body,
      grid=(grid,),
      in_specs=in_specs,
      out_specs=[pl.BlockSpec((BLK, LD), lambda r: (r, 0)),
                 pl.BlockSpec((BLK, 1), lambda r: (r, 0))],
      out_shape=[jax.ShapeDtypeStruct((n, LD), jnp.float32),
                 jax.ShapeDtypeStruct((n, 1), jnp.float32)],
  )(*args)


def _combine_scale(parts, deg, n, d):
  def body(p_r, d_r, o_r):
    o_r[...] = (p_r[0] + p_r[1]) / jnp.maximum(d_r[...], 1e-12)

  return pl.pallas_call(
      body,
      grid=(n // BLK,),
      in_specs=[pl.BlockSpec((2, BLK, d), lambda r: (0, r, 0)),
                pl.BlockSpec((BLK, 1), lambda r: (r, 0))],
      out_specs=pl.BlockSpec((BLK, d), lambda r: (r, 0)),
      out_shape=jax.ShapeDtypeStruct((n, d), jnp.float32),
  )(parts, deg)


def _row_scale(x, deg, n, d):
  def body(x_r, d_r, o_r):
    o_r[...] = x_r[...] / jnp.maximum(d_r[...], 1e-12)

  return pl.pallas_call(
      body,
      grid=(n // BLK,),
      in_specs=[pl.BlockSpec((BLK, d), lambda r: (r, 0)),
                pl.BlockSpec((BLK, 1), lambda r: (r, 0))],
      out_specs=pl.BlockSpec((BLK, d), lambda r: (r, 0)),
      out_shape=jax.ShapeDtypeStruct((n, d), jnp.float32),
  )(x, deg)


def _edge_weights(eW1, eb1):
  wi, wj, wp = eW1[:LD], eW1[LD:2 * LD], eW1[2 * LD:]
  z16 = jnp.zeros((LD, DE - LD), jnp.float32)
  zp = jnp.zeros((PD, DE - LD), jnp.float32)
  zr = jnp.zeros((DE - LD - PD, DE), jnp.float32)
  wa = jnp.concatenate([
      jnp.concatenate([wi, z16], 1),
      jnp.concatenate([-wp, zp], 1),
      zr,
  ], 0)
  wb = jnp.concatenate([
      jnp.concatenate([wj, z16], 1),
      jnp.concatenate([wp, zp], 1),
      zr,
  ], 0)
  ba = jnp.concatenate([eb1, jnp.ones((1,), jnp.float32),
                        jnp.zeros((DE - LD - 1,), jnp.float32)])[None]
  bb = jnp.zeros((1, DE), jnp.float32)
  return wa, ba, wb, bb


def kernel(h, m_ids, m_gs, pos, down_eW1, down_eb1, down_eW2, down_eb2,
           down_nW1, down_nb1, down_nW2, down_nb2, bottom_eW1, bottom_eb1,
           bottom_eW2, bottom_eb2, bottom_nW1, bottom_nb1, bottom_nW2,
           bottom_nb2, up_eW1, up_eb1, up_eW2, up_eb2, up_nW1, up_nb1,
           up_nW2, up_nb2):
  f32 = jnp.float32
  g0 = m_gs[0]
  g1 = m_gs[1]
  i0, j0 = g0[0], g0[1]
  i1, j1 = g1[0], g1[1]
  ids0 = m_ids[0]

  ids_pad = jnp.concatenate(
      [ids0, jnp.full((NP - NC,), N, jnp.int32)]).astype(jnp.int32)
  wmask = jnp.concatenate([ids0[:-1] != ids0[1:],
                           jnp.ones((1,), bool)])
  tgt = jnp.where(wmask, ids0, NF).astype(jnp.int32)
  tgt = jnp.concatenate([tgt, jnp.full((NP - NC,), NF, jnp.int32)])

  pospad = jnp.concatenate(
      [pos[:NP], jnp.zeros((NP, DE - LD - PD), f32)], 1)
  h_pad = jnp.concatenate([h, jnp.zeros((NF - N, LD), f32)], 0)

  nce = (E // 32) // CHE
  def chunked(ix):
    return ix.astype(jnp.int32).reshape(32, nce, CHE)
  i0c, j0c = chunked(i0), chunked(j0)
  i1c, j1c = chunked(i1), chunked(j1)

  relu_pass = _make_relu_pass(DE, NP, E)
  copy_pass_de = _make_copy_pass(DE, NP, E)
  copy_pass_ld = _make_copy_pass(LD, NP, E)
  gather_de = _make_gather(DE, NP, NF)
  scatter_ld = _make_scatter(LD, NP, NT)

  def pad_parts(sp):
    return jnp.concatenate(
        [sp, jnp.zeros((2, NF - NP, sp.shape[-1]), f32)], 1)

  wa, ba, wb, bb = _edge_weights(down_eW1, down_eb1)
  x0 = jnp.concatenate([h[:NP], pospad], 1)
  a0, b0 = _mm_ab(x0, wa, ba, wb, bb)
  s0 = relu_pass(a0, b0, i0c, j0c)
  h1, deg0 = _node_mlp(pad_parts(s0), h_pad, None, down_eW2, down_eb2[None],
                       down_nW1[:LD], down_nW1[LD:], down_nb1[None],
                       down_nW2, down_nb2[None], NF)

  h1ext = jnp.concatenate([h1[:NP], pospad], 1)
  cparts = copy_pass_de(h1ext, j0c, i0c)
  conv = _combine_scale(cparts, deg0[:NP], NP, DE)
  ptbl = jnp.concatenate([conv, jnp.zeros((NF - NP, DE), f32)], 0)
  pooled = gather_de(ptbl, ids_pad)

  wa1, ba1, wb1, bb1 = _edge_weights(bottom_eW1, bottom_eb1)
  a1, b1 = _mm_ab(pooled, wa1, ba1, wb1, bb1)
  s1 = relu_pass(a1, b1, i1c, j1c)
  hc, _ = _node_mlp(s1, pooled[:, :LD], None, bottom_eW2, bottom_eb2[None],
                    bottom_nW1[:LD], bottom_nW1[LD:], bottom_nb1[None],
                    bottom_nW2, bottom_nb2[None], NP)

  hu0 = scatter_ld(hc, tgt)
  hus = _row_scale(hu0[:NP], deg0[:NP], NP, LD)
  uparts = copy_pass_ld(hus, i0c, j0c)
  ones = jnp.ones((NP, 1), f32)
  u = _combine_scale(uparts, ones, NP, LD)
  u_pad = jnp.concatenate([u, jnp.zeros((NF - NP, LD), f32)], 0)

  wa2, ba2, wb2, bb2 = _edge_weights(up_eW1, up_eb1)
  x2 = jnp.concatenate([u, pospad], 1)
  a2, b2 = _mm_ab(x2, wa2, ba2, wb2, bb2)
  s2 = relu_pass(a2, b2, i0c, j0c)
  out, _ = _node_mlp(pad_parts(s2), u_pad, h1, up_eW2, up_eb2[None],
                     up_nW1[:LD], up_nW1[LD:], up_nb1[None],
                     up_nW2, up_nb2[None], NF)
  return out[:N]
